# baseline (speedup 1.0000x reference)
import functools

import jax
import jax.numpy as jnp
from jax import lax
from jax.experimental import pallas as pl
from jax.experimental.pallas import tpu as pltpu
from jax.experimental.pallas import tpu_sc as plsc

N = 10000
NP = 10240
D = 128
H = 128
OUTD = 64
AH = 64
E = 320000
NT = 32
EPT = 10240
CH = 128
NCH = EPT // CH
EPAD = NT * EPT
RPT = NP // 16
GAMMA = -0.1
ZETA = 1.1
RB = 1280


def _dense_body(x_ref, nbW_ref, nbb_ref, sW_ref, sb_ref, aT_ref, aB_ref,
                attb_ref, W_ref, a_ref, b_ref, pa_ref, pb_ref):
    xb = x_ref[...]
    h1 = jnp.maximum(
        jnp.dot(xb, nbW_ref[...], preferred_element_type=jnp.float32)
        + nbb_ref[...], 0.0)
    h2 = jnp.maximum(
        jnp.dot(xb, sW_ref[...], preferred_element_type=jnp.float32)
        + sb_ref[...], 0.0)
    a_ref[...] = (jnp.dot(h1, aT_ref[...], preferred_element_type=jnp.float32)
                  + attb_ref[0, 0])
    b_ref[...] = jnp.dot(h2, aB_ref[...], preferred_element_type=jnp.float32)
    pre = jnp.dot(xb, W_ref[...], preferred_element_type=jnp.float32)
    pa_ref[...] = pre[:, :MF]
    pb_ref[...] = pre[:, MF:]


def _tc_dense(x, nbW, nbb, sW, sb, aT, aB, attb2, W):
    din = x.shape[1]
    full = lambda shape: pl.BlockSpec(shape, lambda i: (0,) * len(shape))
    return pl.pallas_call(
        _dense_body,
        grid=(NP // RB,),
        in_specs=[
            pl.BlockSpec((RB, din), lambda i: (i, 0)),
            full((din, AH)), full((AH,)), full((din, AH)), full((AH,)),
            full((AH, 1)), full((AH, 1)), full((1, 1)), full((din, H)),
        ],
        out_specs=[
            pl.BlockSpec((RB, 1), lambda i: (i, 0)),
            pl.BlockSpec((RB, 1), lambda i: (i, 0)),
            pl.BlockSpec((RB, MF), lambda i: (i, 0)),
            pl.BlockSpec((RB, MF), lambda i: (i, 0)),
        ],
        out_shape=[
            jax.ShapeDtypeStruct((NP, 1), jnp.float32),
            jax.ShapeDtypeStruct((NP, 1), jnp.float32),
            jax.ShapeDtypeStruct((NP, MF), jnp.float32),
            jax.ShapeDtypeStruct((NP, MF), jnp.float32),
        ],
    )(x, nbW, nbb, sW, sb, aT, aB, attb2, W)


def _mid_body(pa_ref, pb_ref, rs_ref, qa_ref, qb_ref, nbW_ref, nbb_ref,
              sW_ref, sb_ref, aT_ref, aB_ref, attb_ref, W_ref, a_ref, b_ref,
              pre1_ref):
    rs = rs_ref[...]
    rsum = rs[0] + rs[1] + 1.0
    dis = lax.rsqrt(rsum)
    pa = pa_ref[...]
    pb = pb_ref[...]
    msg = jnp.concatenate([pa[0] + pa[1], pb[0] + pb[1]], axis=1)
    pre = jnp.concatenate([qa_ref[...], qb_ref[...]], axis=1)
    x1 = jnp.maximum(dis[:, None] * msg + pre / rsum[:, None], 0.0)
    h1 = jnp.maximum(
        jnp.dot(x1, nbW_ref[...], preferred_element_type=jnp.float32)
        + nbb_ref[...], 0.0)
    h2 = jnp.maximum(
        jnp.dot(x1, sW_ref[...], preferred_element_type=jnp.float32)
        + sb_ref[...], 0.0)
    a_ref[...] = (jnp.dot(h1, aT_ref[...], preferred_element_type=jnp.float32)
                  + attb_ref[0, 0])
    b_ref[...] = jnp.dot(h2, aB_ref[...], preferred_element_type=jnp.float32)
    pre1_ref[...] = jnp.dot(x1, W_ref[...], preferred_element_type=jnp.float32)


def _tc_mid(partsA, partsB, rs, preA, preB, nbW, nbb, sW, sb, aT, aB,
            attb2, W):
    full = lambda shape: pl.BlockSpec(shape, lambda i: (0,) * len(shape))
    return pl.pallas_call(
        _mid_body,
        grid=(NP // RB,),
        in_specs=[
            pl.BlockSpec((2, RB, MF), lambda i: (0, i, 0)),
            pl.BlockSpec((2, RB, MF), lambda i: (0, i, 0)),
            pl.BlockSpec((2, RB), lambda i: (0, i)),
            pl.BlockSpec((RB, MF), lambda i: (i, 0)),
            pl.BlockSpec((RB, MF), lambda i: (i, 0)),
            full((H, AH)), full((AH,)), full((H, AH)), full((AH,)),
            full((AH, 1)), full((AH, 1)), full((1, 1)), full((H, OUTD)),
        ],
        out_specs=[
            pl.BlockSpec((RB, 1), lambda i: (i, 0)),
            pl.BlockSpec((RB, 1), lambda i: (i, 0)),
            pl.BlockSpec((RB, OUTD), lambda i: (i, 0)),
        ],
        out_shape=[
            jax.ShapeDtypeStruct((NP, 1), jnp.float32),
            jax.ShapeDtypeStruct((NP, 1), jnp.float32),
            jax.ShapeDtypeStruct((NP, OUTD), jnp.float32),
        ],
    )(partsA, partsB, rs, preA, preB, nbW, nbb, sW, sb, aT, aB, attb2, W)


def _fin_body(p_ref, rs_ref, pre_ref, o_ref):
    rs = rs_ref[...]
    rsum = rs[0] + rs[1] + 1.0
    dis = lax.rsqrt(rsum)
    p = p_ref[...]
    o_ref[...] = dis[:, None] * (p[0] + p[1]) + pre_ref[...] / rsum[:, None]


def _tc_fin(parts, rs, pre):
    return pl.pallas_call(
        _fin_body,
        grid=(NP // RB,),
        in_specs=[
            pl.BlockSpec((2, RB, OUTD), lambda i: (0, i, 0)),
            pl.BlockSpec((2, RB), lambda i: (0, i)),
            pl.BlockSpec((RB, OUTD), lambda i: (i, 0)),
        ],
        out_specs=pl.BlockSpec((RB, OUTD), lambda i: (i, 0)),
        out_shape=jax.ShapeDtypeStruct((NP, OUTD), jnp.float32),
    )(parts, rs, pre)


_MESH = plsc.VectorSubcoreMesh(core_axis_name="c", subcore_axis_name="s")
_SC_PARAMS = pltpu.CompilerParams(needs_layout_passes=False,
                                  use_tc_tiling_on_sc=False)


def _edges_body(a_h, b_h, ri_h, ci_h, mask_h, rs_h,
                a_v, b_v, ri_v, ci_v, m_v, rsum_v, red_v, stag):
    c = lax.axis_index("c")
    s = lax.axis_index("s")
    tb = c * 16 + s
    pltpu.sync_copy(a_h, a_v)
    pltpu.sync_copy(b_h, b_v)
    pltpu.sync_copy(ri_h.at[tb], ri_v)
    pltpu.sync_copy(ci_h.at[tb], ci_v)
    zero16 = jnp.zeros((16,), jnp.float32)

    def zbody(i, _):
        rsum_v[pl.ds(i * 16, 16)] = zero16
        return 0
    lax.fori_loop(0, NP // 16, zbody, 0)

    def ebody(j, _):
        for u in range(2):
            sl = pl.ds(j * 32 + u * 16, 16)
            r16 = ri_v[sl]
            c16 = ci_v[sl]
            la = plsc.load_gather(a_v, [r16]) + plsc.load_gather(b_v, [c16])
            gate = 1.0 / (1.0 + jnp.exp(-la))
            m = jnp.clip(gate * (ZETA - GAMMA) + GAMMA, 0.0, 1.0)
            m_v[sl] = m
            plsc.addupdate_scatter(rsum_v, [r16], m)
        return 0
    lax.fori_loop(0, EPT // 32, ebody, 0)

    pltpu.sync_copy(m_v, mask_h.at[tb])
    pltpu.sync_copy(rsum_v, stag.at[s])
    plsc.subcore_barrier()
    for r in range(16):
        pltpu.sync_copy(stag.at[r, pl.ds(s * RPT, RPT)], red_v.at[r])

    def rbody(k, _):
        sl = pl.ds(k * 16, 16)
        t = red_v[0, sl]
        for r in range(1, 16):
            t = t + red_v[r, sl]
        rsum_v[sl] = t
        return 0
    lax.fori_loop(0, RPT // 16, rbody, 0)
    pltpu.sync_copy(rsum_v.at[pl.ds(0, RPT)], rs_h.at[c, pl.ds(s * RPT, RPT)])


def _sc_edges(a_tab, b_tab, ri2, ci2):
    kern = pl.kernel(
        _edges_body,
        out_type=[
            jax.ShapeDtypeStruct((NT, EPT), jnp.float32),
            jax.ShapeDtypeStruct((2, NP), jnp.float32),
        ],
        mesh=_MESH,
        scratch_types=[
            pltpu.VMEM((NP,), jnp.float32),
            pltpu.VMEM((NP,), jnp.float32),
            pltpu.VMEM((EPT,), jnp.int32),
            pltpu.VMEM((EPT,), jnp.int32),
            pltpu.VMEM((EPT,), jnp.float32),
            pltpu.VMEM((NP,), jnp.float32),
            pltpu.VMEM((16, RPT), jnp.float32),
            pltpu.VMEM_SHARED((16, NP), jnp.float32),
        ],
        compiler_params=_SC_PARAMS,
    )
    return kern(a_tab, b_tab, ri2, ci2)


MF = 64
MCH = 128
MNCH = EPT // MCH
MNBUF = 2
_PK = plsc.PackFormat.INTERLEAVED


def _msg_body(pre_h, rs_h, mask_h, ri_h, ci_h, out_h, ri_v, ci_v, d_v, *rest):
    gbufs = rest[:MNBUF]
    sbufs = rest[MNBUF:2 * MNBUF]
    mbufs = rest[2 * MNBUF:3 * MNBUF]
    gsems = rest[3 * MNBUF:4 * MNBUF]
    ssems = rest[4 * MNBUF:5 * MNBUF]
    msems = rest[5 * MNBUF:6 * MNBUF]
    tab = rest[6 * MNBUF]
    acc = rest[6 * MNBUF + 1]
    c = lax.axis_index("c")
    s = lax.axis_index("s")
    tb = c * 16 + s
    pltpu.sync_copy(ri_h.at[tb], ri_v)
    pltpu.sync_copy(ci_h.at[tb], ci_v)
    base = s * RPT
    pltpu.sync_copy(rs_h.at[0, pl.ds(base, RPT)], d_v.at[pl.ds(0, RPT)])
    pltpu.sync_copy(rs_h.at[1, pl.ds(base, RPT)], d_v.at[pl.ds(RPT, RPT)])

    def dbody(i, _):
        sl = pl.ds(i * 16, 16)
        xs = d_v[sl] + d_v[pl.ds(RPT + i * 16, 16)] + 1.0
        ii = plsc.bitcast(xs, jnp.int32)
        ii = jnp.int32(0x5F3759DF) - (ii >> 1)
        y = plsc.bitcast(ii, jnp.float32)
        for _ in range(3):
            y = y * (1.5 - 0.5 * xs * y * y)
        d_v[sl] = y
        return 0
    lax.fori_loop(0, RPT // 16, dbody, 0)

    zero16 = jnp.zeros((16,), jnp.float32)
    s0 = sbufs[0]
    g0 = gbufs[0]
    for k in range(RPT // MCH):
        pltpu.sync_copy(pre_h.at[pl.ds(base + k * MCH, MCH)], s0)

        def stb(e, _):
            ws = plsc.load_gather(d_v, [jnp.full((16,), k * MCH + e,
                                                 jnp.int32)])
            for f2 in range(MF // 32):
                va = s0[e, pl.ds(f2 * 32, 16)] * ws
                vb = s0[e, pl.ds(f2 * 32 + 16, 16)] * ws
                g0[e, pl.ds(f2 * 32, 32)] = plsc.pack(va, vb, format=_PK)
            return 0
        lax.fori_loop(0, MCH, stb, 0)
        pltpu.sync_copy(g0, tab.at[pl.ds(base + k * MCH, MCH)])

    def zb(e, _):
        for f in range(MF // 16):
            s0[e, pl.ds(f * 16, 16)] = zero16
        return 0
    lax.fori_loop(0, MCH, zb, 0)
    for k in range(RPT // MCH):
        pltpu.sync_copy(s0, acc.at[pl.ds(base + k * MCH, MCH)])
    plsc.subcore_barrier()

    def gstart(jn, b):
        pltpu.make_async_copy(tab.at[ci_v.at[jn]], gbufs[b],
                              gsems[b]).start()
        pltpu.make_async_copy(mask_h.at[tb, pl.ds(jn * MCH, MCH)],
                              mbufs[b], msems[b]).start()

    def gwait(b):
        pltpu.make_async_copy(tab.at[ci_v.at[0]], gbufs[b],
                              gsems[b]).wait()
        pltpu.make_async_copy(mask_h.at[tb, pl.ds(0, MCH)],
                              mbufs[b], msems[b]).wait()

    def sstart(j, b):
        pltpu.make_async_copy(sbufs[b], acc.at[ri_v.at[j]],
                              ssems[b]).start(add=True)

    def swait(b):
        pltpu.make_async_copy(sbufs[b], acc.at[ri_v.at[0]],
                              ssems[b]).wait()

    def scale(j, b):
        gb = gbufs[b]
        sb = sbufs[b]
        mb = mbufs[b]
        UNR = 2

        def sc4(e4, _):
            e = e4 * UNR
            ws = [plsc.load_gather(mb, [jnp.full((16,), e + u, jnp.int32)])
                  for u in range(UNR)]
            for u in range(UNR):
                for f2 in range(MF // 32):
                    v32 = gb[e + u, pl.ds(f2 * 32, 32)]
                    va, vb = plsc.unpack(v32, format=_PK)
                    sb[e + u, pl.ds(f2 * 32, 16)] = va * ws[u]
                    sb[e + u, pl.ds(f2 * 32 + 16, 16)] = vb * ws[u]
            return 0
        lax.fori_loop(0, MCH // UNR, sc4, 0)

    def substep(j, b):
        gwait(b)
        scale(j, b)
        sstart(j, b)

    for b in range(MNBUF - 1):
        gstart(b, b)
    for b in range(MNBUF):
        substep(b, b)
        if b > 0:
            swait((b - 1) % MNBUF)
        if b + MNBUF - 1 < MNCH:
            gstart(b + MNBUF - 1, (b - 1) % MNBUF)

    def round_body(p, _):
        for b in range(MNBUF):
            j = p * MNBUF + b
            substep(j, b)
            swait((b - 1) % MNBUF)
            gstart(j + MNBUF - 1, (b - 1) % MNBUF)
        return 0
    lax.fori_loop(1, MNCH // MNBUF - 1, round_body, 0)

    for j in range((MNCH // MNBUF - 1) * MNBUF, MNCH):
        b = j % MNBUF
        substep(j, b)
        swait((b - 1) % MNBUF)
        if j + MNBUF - 1 < MNCH:
            gstart(j + MNBUF - 1, (b - 1) % MNBUF)
    swait((MNCH - 1) % MNBUF)

    plsc.subcore_barrier()
    for k in range(RPT // MCH):
        sl = pl.ds(s * RPT + k * MCH, MCH)
        pltpu.sync_copy(acc.at[sl], out_h.at[c, sl])


def _sc_msg(pre, rs, mask, ri3, ci3):
    kern = pl.kernel(
        _msg_body,
        out_type=jax.ShapeDtypeStruct((2, NP, MF), jnp.float32),
        mesh=_MESH,
        scratch_types=[
            pltpu.VMEM((MNCH, MCH), jnp.int32),
            pltpu.VMEM((MNCH, MCH), jnp.int32),
            pltpu.VMEM((2 * RPT,), jnp.float32),
        ] + [pltpu.VMEM((MCH, MF), jnp.bfloat16) for _ in range(MNBUF)]
          + [pltpu.VMEM((MCH, MF), jnp.float32) for _ in range(MNBUF)]
          + [pltpu.VMEM((MCH,), jnp.float32) for _ in range(MNBUF)]
          + [pltpu.SemaphoreType.DMA for _ in range(3 * MNBUF)]
          + [pltpu.VMEM_SHARED((NP, MF), jnp.bfloat16)
             , pltpu.VMEM_SHARED((NP, MF), jnp.float32)],
        compiler_params=_SC_PARAMS,
    )
    return kern(pre, rs, mask, ri3, ci3)


def kernel(x, edge_index, W0, W1,
           nbW0, nbb0, selfW0, selfb0, attW0, attb0,
           nbW1, nbb1, selfW1, selfb1, attW1, attb1):
    row = edge_index[0].astype(jnp.int32)
    col = edge_index[1].astype(jnp.int32)
    pad = jnp.full((EPAD - E,), N, jnp.int32)
    rp = jnp.concatenate([row, pad])
    cp = jnp.concatenate([col, pad])
    ri2 = rp.reshape(NT, EPT)
    ci2 = cp.reshape(NT, EPT)
    ri3 = rp.reshape(NT, MNCH, MCH)
    ci3 = cp.reshape(NT, MNCH, MCH)
    xp = jnp.concatenate([x, jnp.zeros((NP - N, D), jnp.float32)])

    a0, b0, preA, preB = _tc_dense(xp, nbW0, nbb0, selfW0, selfb0,
                                   attW0[:AH], attW0[AH:],
                                   attb0.reshape(1, 1), W0)
    mask0, rs0 = _sc_edges(a0.reshape(NP), b0.reshape(NP), ri2, ci2)
    partsA = _sc_msg(preA, rs0, mask0, ri3, ci3)
    partsB = _sc_msg(preB, rs0, mask0, ri3, ci3)

    a1, b1, pre1 = _tc_mid(partsA, partsB, rs0, preA, preB, nbW1, nbb1,
                           selfW1, selfb1, attW1[:AH], attW1[AH:],
                           attb1.reshape(1, 1), W1)
    mask1, rs1 = _sc_edges(a1.reshape(NP), b1.reshape(NP), ri2, ci2)
    parts1 = _sc_msg(pre1, rs1, mask1, ri3, ci3)
    out = _tc_fin(parts1, rs1, pre1)
    return out[:N]


# speedup vs baseline: 1.5983x; 1.5983x over previous
import functools

import jax
import jax.numpy as jnp
from jax import lax
from jax.experimental import pallas as pl
from jax.experimental.pallas import tpu as pltpu
from jax.experimental.pallas import tpu_sc as plsc

N = 10000
NP = 10240
D = 128
H = 128
OUTD = 64
AH = 64
E = 320000
NT = 32
EPT = 10240
CH = 128
NCH = EPT // CH
EPAD = NT * EPT
RPT = NP // 16
GAMMA = -0.1
ZETA = 1.1
RB = 1280


def _dense_body(x_ref, nbW_ref, nbb_ref, sW_ref, sb_ref, aT_ref, aB_ref,
                attb_ref, W_ref, a_ref, b_ref, pa_ref, pb_ref):
    xb = x_ref[...]
    h1 = jnp.maximum(
        jnp.dot(xb, nbW_ref[...], preferred_element_type=jnp.float32)
        + nbb_ref[...], 0.0)
    h2 = jnp.maximum(
        jnp.dot(xb, sW_ref[...], preferred_element_type=jnp.float32)
        + sb_ref[...], 0.0)
    a_ref[...] = (jnp.dot(h1, aT_ref[...], preferred_element_type=jnp.float32)
                  + attb_ref[0, 0])
    b_ref[...] = jnp.dot(h2, aB_ref[...], preferred_element_type=jnp.float32)
    pre = jnp.dot(xb, W_ref[...], preferred_element_type=jnp.float32)
    pa_ref[...] = pre[:, :MF]
    pb_ref[...] = pre[:, MF:]


def _tc_dense(x, nbW, nbb, sW, sb, aT, aB, attb2, W):
    din = x.shape[1]
    full = lambda shape: pl.BlockSpec(shape, lambda i: (0,) * len(shape))
    return pl.pallas_call(
        _dense_body,
        grid=(NP // RB,),
        in_specs=[
            pl.BlockSpec((RB, din), lambda i: (i, 0)),
            full((din, AH)), full((AH,)), full((din, AH)), full((AH,)),
            full((AH, 1)), full((AH, 1)), full((1, 1)), full((din, H)),
        ],
        out_specs=[
            pl.BlockSpec((RB, 1), lambda i: (i, 0)),
            pl.BlockSpec((RB, 1), lambda i: (i, 0)),
            pl.BlockSpec((RB, MF), lambda i: (i, 0)),
            pl.BlockSpec((RB, MF), lambda i: (i, 0)),
        ],
        out_shape=[
            jax.ShapeDtypeStruct((NP, 1), jnp.float32),
            jax.ShapeDtypeStruct((NP, 1), jnp.float32),
            jax.ShapeDtypeStruct((NP, MF), jnp.float32),
            jax.ShapeDtypeStruct((NP, MF), jnp.float32),
        ],
    )(x, nbW, nbb, sW, sb, aT, aB, attb2, W)


def _mid_body(pa_ref, pb_ref, rs_ref, qa_ref, qb_ref, nbW_ref, nbb_ref,
              sW_ref, sb_ref, aT_ref, aB_ref, attb_ref, W_ref, a_ref, b_ref,
              pre1_ref):
    rs = rs_ref[...]
    rsum = rs[0] + rs[1] + 1.0
    dis = lax.rsqrt(rsum)
    pa = pa_ref[...]
    pb = pb_ref[...]
    msg = jnp.concatenate([pa[0] + pa[1], pb[0] + pb[1]], axis=1)
    pre = jnp.concatenate([qa_ref[...], qb_ref[...]], axis=1)
    x1 = jnp.maximum(dis[:, None] * msg + pre / rsum[:, None], 0.0)
    h1 = jnp.maximum(
        jnp.dot(x1, nbW_ref[...], preferred_element_type=jnp.float32)
        + nbb_ref[...], 0.0)
    h2 = jnp.maximum(
        jnp.dot(x1, sW_ref[...], preferred_element_type=jnp.float32)
        + sb_ref[...], 0.0)
    a_ref[...] = (jnp.dot(h1, aT_ref[...], preferred_element_type=jnp.float32)
                  + attb_ref[0, 0])
    b_ref[...] = jnp.dot(h2, aB_ref[...], preferred_element_type=jnp.float32)
    pre1_ref[...] = jnp.dot(x1, W_ref[...], preferred_element_type=jnp.float32)


def _tc_mid(partsA, partsB, rs, preA, preB, nbW, nbb, sW, sb, aT, aB,
            attb2, W):
    full = lambda shape: pl.BlockSpec(shape, lambda i: (0,) * len(shape))
    return pl.pallas_call(
        _mid_body,
        grid=(NP // RB,),
        in_specs=[
            pl.BlockSpec((2, RB, MF), lambda i: (0, i, 0)),
            pl.BlockSpec((2, RB, MF), lambda i: (0, i, 0)),
            pl.BlockSpec((2, RB), lambda i: (0, i)),
            pl.BlockSpec((RB, MF), lambda i: (i, 0)),
            pl.BlockSpec((RB, MF), lambda i: (i, 0)),
            full((H, AH)), full((AH,)), full((H, AH)), full((AH,)),
            full((AH, 1)), full((AH, 1)), full((1, 1)), full((H, OUTD)),
        ],
        out_specs=[
            pl.BlockSpec((RB, 1), lambda i: (i, 0)),
            pl.BlockSpec((RB, 1), lambda i: (i, 0)),
            pl.BlockSpec((RB, OUTD), lambda i: (i, 0)),
        ],
        out_shape=[
            jax.ShapeDtypeStruct((NP, 1), jnp.float32),
            jax.ShapeDtypeStruct((NP, 1), jnp.float32),
            jax.ShapeDtypeStruct((NP, OUTD), jnp.float32),
        ],
    )(partsA, partsB, rs, preA, preB, nbW, nbb, sW, sb, aT, aB, attb2, W)


def _fin_body(p_ref, rs_ref, pre_ref, o_ref):
    rs = rs_ref[...]
    rsum = rs[0] + rs[1] + 1.0
    dis = lax.rsqrt(rsum)
    p = p_ref[...]
    o_ref[...] = dis[:, None] * (p[0] + p[1]) + pre_ref[...] / rsum[:, None]


def _tc_fin(parts, rs, pre):
    return pl.pallas_call(
        _fin_body,
        grid=(NP // RB,),
        in_specs=[
            pl.BlockSpec((2, RB, OUTD), lambda i: (0, i, 0)),
            pl.BlockSpec((2, RB), lambda i: (0, i)),
            pl.BlockSpec((RB, OUTD), lambda i: (i, 0)),
        ],
        out_specs=pl.BlockSpec((RB, OUTD), lambda i: (i, 0)),
        out_shape=jax.ShapeDtypeStruct((NP, OUTD), jnp.float32),
    )(parts, rs, pre)


_MESH = plsc.VectorSubcoreMesh(core_axis_name="c", subcore_axis_name="s")
_SC_PARAMS = pltpu.CompilerParams(needs_layout_passes=False,
                                  use_tc_tiling_on_sc=False)


def _edges_body(a_h, b_h, ri_h, ci_h, mask_h, rs_h,
                a_v, b_v, ri_v, ci_v, m_v, rsum_v, red_v, stag):
    c = lax.axis_index("c")
    s = lax.axis_index("s")
    tb = c * 16 + s
    pltpu.sync_copy(a_h, a_v)
    pltpu.sync_copy(b_h, b_v)
    pltpu.sync_copy(ri_h.at[tb], ri_v)
    pltpu.sync_copy(ci_h.at[tb], ci_v)
    zero16 = jnp.zeros((16,), jnp.float32)

    def zbody(i, _):
        rsum_v[pl.ds(i * 16, 16)] = zero16
        return 0
    lax.fori_loop(0, NP // 16, zbody, 0)

    def ebody(j, _):
        for u in range(2):
            sl = pl.ds(j * 32 + u * 16, 16)
            r16 = ri_v[sl]
            c16 = ci_v[sl]
            la = plsc.load_gather(a_v, [r16]) + plsc.load_gather(b_v, [c16])
            gate = 1.0 / (1.0 + jnp.exp(-la))
            m = jnp.clip(gate * (ZETA - GAMMA) + GAMMA, 0.0, 1.0)
            m_v[sl] = m
            plsc.addupdate_scatter(rsum_v, [r16], m)
        return 0
    lax.fori_loop(0, EPT // 32, ebody, 0)

    pltpu.sync_copy(m_v, mask_h.at[tb])
    pltpu.sync_copy(rsum_v, stag.at[s])
    plsc.subcore_barrier()
    for r in range(16):
        pltpu.sync_copy(stag.at[r, pl.ds(s * RPT, RPT)], red_v.at[r])

    def rbody(k, _):
        sl = pl.ds(k * 16, 16)
        t = red_v[0, sl]
        for r in range(1, 16):
            t = t + red_v[r, sl]
        rsum_v[sl] = t
        return 0
    lax.fori_loop(0, RPT // 16, rbody, 0)
    pltpu.sync_copy(rsum_v.at[pl.ds(0, RPT)], rs_h.at[c, pl.ds(s * RPT, RPT)])


def _sc_edges(a_tab, b_tab, ri2, ci2):
    kern = pl.kernel(
        _edges_body,
        out_type=[
            jax.ShapeDtypeStruct((NT, EPT), jnp.float32),
            jax.ShapeDtypeStruct((2, NP), jnp.float32),
        ],
        mesh=_MESH,
        scratch_types=[
            pltpu.VMEM((NP,), jnp.float32),
            pltpu.VMEM((NP,), jnp.float32),
            pltpu.VMEM((EPT,), jnp.int32),
            pltpu.VMEM((EPT,), jnp.int32),
            pltpu.VMEM((EPT,), jnp.float32),
            pltpu.VMEM((NP,), jnp.float32),
            pltpu.VMEM((16, RPT), jnp.float32),
            pltpu.VMEM_SHARED((16, NP), jnp.float32),
        ],
        compiler_params=_SC_PARAMS,
    )
    return kern(a_tab, b_tab, ri2, ci2)


MF = 64
MCH = 128
MNCH = EPT // MCH
MNBUF = 3


def _msg_body(NPH, *refs):
    pres = refs[:NPH]
    rs_h, mask_h, ri_h, ci_h = refs[NPH:NPH + 4]
    outs = refs[NPH + 4:2 * NPH + 4]
    rest = refs[2 * NPH + 4:]
    ri_v, ci_v, d_v = rest[:3]
    rest = rest[3:]
    gbufs = rest[:MNBUF]
    mbufs = rest[MNBUF:2 * MNBUF]
    gsems = rest[2 * MNBUF:3 * MNBUF]
    ssems = rest[3 * MNBUF:4 * MNBUF]
    msems = rest[4 * MNBUF:5 * MNBUF]
    tab = rest[5 * MNBUF]
    acc = rest[5 * MNBUF + 1]
    c = lax.axis_index("c")
    s = lax.axis_index("s")
    tb = c * 16 + s
    pltpu.sync_copy(ri_h.at[tb], ri_v)
    pltpu.sync_copy(ci_h.at[tb], ci_v)
    base = s * RPT
    pltpu.sync_copy(rs_h.at[0, pl.ds(base, RPT)], d_v.at[pl.ds(0, RPT)])
    pltpu.sync_copy(rs_h.at[1, pl.ds(base, RPT)], d_v.at[pl.ds(RPT, RPT)])

    def dbody(i, _):
        sl = pl.ds(i * 16, 16)
        xs = d_v[sl] + d_v[pl.ds(RPT + i * 16, 16)] + 1.0
        ii = plsc.bitcast(xs, jnp.int32)
        ii = jnp.int32(0x5F3759DF) - (ii >> 1)
        y = plsc.bitcast(ii, jnp.float32)
        for _ in range(3):
            y = y * (1.5 - 0.5 * xs * y * y)
        d_v[sl] = y
        return 0
    lax.fori_loop(0, RPT // 16, dbody, 0)

    for ph in range(NPH):
        _msg_phase(pres[ph], outs[ph], mask_h, ri_v, ci_v, d_v, gbufs,
                   mbufs, gsems, ssems, msems, tab, acc, c, s, tb, base)


def _msg_phase(pre_h, out_h, mask_h, ri_v, ci_v, d_v, gbufs, mbufs,
               gsems, ssems, msems, tab, acc, c, s, tb, base):
    zero16 = jnp.zeros((16,), jnp.float32)
    g0 = gbufs[0]
    for k in range(RPT // MCH):
        pltpu.sync_copy(pre_h.at[pl.ds(base + k * MCH, MCH)], g0)

        def stb(e, _):
            ws = plsc.load_gather(d_v, [jnp.full((16,), k * MCH + e,
                                                 jnp.int32)])
            for f in range(MF // 16):
                sl = pl.ds(f * 16, 16)
                g0[e, sl] = g0[e, sl] * ws
            return 0
        lax.fori_loop(0, MCH, stb, 0)
        pltpu.sync_copy(g0, tab.at[pl.ds(base + k * MCH, MCH)])

    def zb(e, _):
        for f in range(MF // 16):
            g0[e, pl.ds(f * 16, 16)] = zero16
        return 0
    lax.fori_loop(0, MCH, zb, 0)
    for k in range(RPT // MCH):
        pltpu.sync_copy(g0, acc.at[pl.ds(base + k * MCH, MCH)])
    plsc.subcore_barrier()

    def gstart(jn, b):
        pltpu.make_async_copy(tab.at[ci_v.at[jn]], gbufs[b],
                              gsems[b]).start()
        pltpu.make_async_copy(mask_h.at[tb, pl.ds(jn * MCH, MCH)],
                              mbufs[b], msems[b]).start()

    def gwait(b):
        pltpu.make_async_copy(tab.at[ci_v.at[0]], gbufs[b],
                              gsems[b]).wait()
        pltpu.make_async_copy(mask_h.at[tb, pl.ds(0, MCH)],
                              mbufs[b], msems[b]).wait()

    def sstart(j, b):
        pltpu.make_async_copy(gbufs[b], acc.at[ri_v.at[j]],
                              ssems[b]).start(add=True)

    def swait(b):
        pltpu.make_async_copy(gbufs[b], acc.at[ri_v.at[0]],
                              ssems[b]).wait()

    def scale(j, b):
        gb = gbufs[b]
        mb = mbufs[b]
        UNR = 4

        def sc4(e4, _):
            e = e4 * UNR
            ws = [plsc.load_gather(mb, [jnp.full((16,), e + u, jnp.int32)])
                  for u in range(UNR)]
            for u in range(UNR):
                for f in range(MF // 16):
                    sl = pl.ds(f * 16, 16)
                    gb[e + u, sl] = gb[e + u, sl] * ws[u]
            return 0
        lax.fori_loop(0, MCH // UNR, sc4, 0)

    def substep(j, b):
        gwait(b)
        scale(j, b)
        sstart(j, b)

    for b in range(MNBUF - 1):
        gstart(b, b)
    for b in range(MNBUF):
        substep(b, b)
        if b > 0:
            swait((b - 1) % MNBUF)
        if b + MNBUF - 1 < MNCH:
            gstart(b + MNBUF - 1, (b - 1) % MNBUF)

    def round_body(p, _):
        for b in range(MNBUF):
            j = p * MNBUF + b
            substep(j, b)
            swait((b - 1) % MNBUF)
            gstart(j + MNBUF - 1, (b - 1) % MNBUF)
        return 0
    lax.fori_loop(1, MNCH // MNBUF - 1, round_body, 0)

    for j in range((MNCH // MNBUF - 1) * MNBUF, MNCH):
        b = j % MNBUF
        substep(j, b)
        swait((b - 1) % MNBUF)
        if j + MNBUF - 1 < MNCH:
            gstart(j + MNBUF - 1, (b - 1) % MNBUF)
    swait((MNCH - 1) % MNBUF)

    plsc.subcore_barrier()
    for k in range(RPT // MCH):
        sl = pl.ds(s * RPT + k * MCH, MCH)
        pltpu.sync_copy(acc.at[sl], out_h.at[c, sl])


def _sc_msg(pre_list, rs, mask, ri3, ci3):
    NPH = len(pre_list)
    kern = pl.kernel(
        functools.partial(_msg_body, NPH),
        out_type=[jax.ShapeDtypeStruct((2, NP, MF), jnp.float32)
                  for _ in range(NPH)],
        mesh=_MESH,
        scratch_types=[
            pltpu.VMEM((MNCH, MCH), jnp.int32),
            pltpu.VMEM((MNCH, MCH), jnp.int32),
            pltpu.VMEM((2 * RPT,), jnp.float32),
        ] + [pltpu.VMEM((MCH, MF), jnp.float32) for _ in range(MNBUF)]
          + [pltpu.VMEM((MCH,), jnp.float32) for _ in range(MNBUF)]
          + [pltpu.SemaphoreType.DMA for _ in range(3 * MNBUF)]
          + [pltpu.VMEM_SHARED((NP, MF), jnp.float32)
             , pltpu.VMEM_SHARED((NP, MF), jnp.float32)],
        compiler_params=_SC_PARAMS,
    )
    return kern(*pre_list, rs, mask, ri3, ci3)


def kernel(x, edge_index, W0, W1,
           nbW0, nbb0, selfW0, selfb0, attW0, attb0,
           nbW1, nbb1, selfW1, selfb1, attW1, attb1):
    row = edge_index[0].astype(jnp.int32)
    col = edge_index[1].astype(jnp.int32)
    pad = jnp.full((EPAD - E,), N, jnp.int32)
    rp = jnp.concatenate([row, pad])
    cp = jnp.concatenate([col, pad])
    ri2 = rp.reshape(NT, EPT)
    ci2 = cp.reshape(NT, EPT)
    ri3 = rp.reshape(NT, MNCH, MCH)
    ci3 = cp.reshape(NT, MNCH, MCH)
    xp = jnp.concatenate([x, jnp.zeros((NP - N, D), jnp.float32)])

    a0, b0, preA, preB = _tc_dense(xp, nbW0, nbb0, selfW0, selfb0,
                                   attW0[:AH], attW0[AH:],
                                   attb0.reshape(1, 1), W0)
    mask0, rs0 = _sc_edges(a0.reshape(NP), b0.reshape(NP), ri2, ci2)
    partsA, partsB = _sc_msg([preA, preB], rs0, mask0, ri3, ci3)

    a1, b1, pre1 = _tc_mid(partsA, partsB, rs0, preA, preB, nbW1, nbb1,
                           selfW1, selfb1, attW1[:AH], attW1[AH:],
                           attb1.reshape(1, 1), W1)
    mask1, rs1 = _sc_edges(a1.reshape(NP), b1.reshape(NP), ri2, ci2)
    (parts1,) = _sc_msg([pre1], rs1, mask1, ri3, ci3)
    out = _tc_fin(parts1, rs1, pre1)
    return out[:N]
